# baseline (speedup 1.0000x reference)
import functools

import jax
import jax.numpy as jnp
from jax import lax
from jax.experimental import pallas as pl
from jax.experimental.pallas import tpu as pltpu

B, T, EMB = 2, 2048, 768
HEADS, DEPTH = 12, 2
BUCKET, NHASH, FF_CHUNKS = 64, 4, 16
D = EMB // HEADS
BN = T // BUCKET
CHUNKS = BN * NHASH
R_ = B * HEADS
RSZ = BN // 2
NS = NHASH * T
CPB = 8


def _proj_body(x_ref, wk_ref, wv_ref, qk_ref, v_ref):
    x = x_ref[0]
    qk_ref[0] = jnp.dot(x, wk_ref[0], preferred_element_type=jnp.float32)
    v_ref[0] = jnp.dot(x, wv_ref[0], preferred_element_type=jnp.float32)


def _proj(x2, Wk, Wv):
    TB = 512
    Wkh = Wk.reshape(EMB, HEADS, D).transpose(1, 0, 2)
    Wvh = Wv.reshape(EMB, HEADS, D).transpose(1, 0, 2)
    return pl.pallas_call(
        _proj_body,
        grid=(R_, T // TB),
        in_specs=[
            pl.BlockSpec((1, TB, EMB), lambda r, t: (r // HEADS, t, 0)),
            pl.BlockSpec((1, EMB, D), lambda r, t: (r % HEADS, 0, 0)),
            pl.BlockSpec((1, EMB, D), lambda r, t: (r % HEADS, 0, 0)),
        ],
        out_specs=[
            pl.BlockSpec((1, TB, D), lambda r, t: (r, t, 0)),
            pl.BlockSpec((1, TB, D), lambda r, t: (r, t, 0)),
        ],
        out_shape=[
            jax.ShapeDtypeStruct((R_, T, D), jnp.float32),
            jax.ShapeDtypeStruct((R_, T, D), jnp.float32),
        ],
    )(x2, Wkh, Wvh)


def _hashsort_body(qk_ref, rcat_ref, undo_ref, oh_ref, cum_ref):
    qk = qk_ref[0]
    xr = jnp.dot(qk, rcat_ref[0], preferred_element_type=jnp.float32)
    il = lax.broadcasted_iota(jnp.int32, (T, BN), 1)
    for h in range(NHASH):
        sub = xr[:, h * BN:(h + 1) * BN]
        m = jnp.max(sub, axis=1, keepdims=True)
        idx = jnp.min(jnp.where(sub == m, il, BN + 1), axis=1, keepdims=True)
        oh_ref[:, h * BN:(h + 1) * BN] = (il == idx).astype(jnp.float32)
    TB = 256
    r_i = lax.broadcasted_iota(jnp.int32, (TB, TB), 0)
    c_i = lax.broadcasted_iota(jnp.int32, (TB, TB), 1)
    L = (c_i <= r_i).astype(jnp.float32)
    carry = jnp.zeros((1, NHASH * BN), jnp.float32)
    for b in range(T // TB):
        blk = oh_ref[b * TB:(b + 1) * TB, :]
        inc = jnp.dot(L, blk, preferred_element_type=jnp.float32)
        cum_ref[b * TB:(b + 1) * TB, :] = inc + carry
        carry = carry + inc[TB - 1:TB, :]
    g_r = lax.broadcasted_iota(jnp.int32, (NHASH * BN, NHASH * BN), 0)
    g_c = lax.broadcasted_iota(jnp.int32, (NHASH * BN, NHASH * BN), 1)
    M = ((g_r // BN == g_c // BN) & (g_r < g_c)).astype(jnp.float32)
    offs = jnp.dot(carry, M, preferred_element_type=jnp.float32)
    cols = []
    for h in range(NHASH):
        oh_h = oh_ref[:, h * BN:(h + 1) * BN]
        cum_h = cum_ref[:, h * BN:(h + 1) * BN]
        rank_incl = jnp.sum(cum_h * oh_h, axis=1, keepdims=True)
        offpick = jnp.sum(offs[:, h * BN:(h + 1) * BN] * oh_h, axis=1, keepdims=True)
        dest = rank_incl - 1.0 + offpick + float(T) * h
        cols.append(dest.astype(jnp.int32))
    undo_ref[0] = jnp.concatenate(cols, axis=1)


def _hashsort(qkh, Rcat):
    return pl.pallas_call(
        _hashsort_body,
        grid=(R_,),
        in_specs=[
            pl.BlockSpec((1, T, D), lambda r: (r, 0, 0)),
            pl.BlockSpec((1, D, NHASH * BN), lambda r: (r, 0, 0)),
        ],
        out_specs=pl.BlockSpec((1, T, NHASH), lambda r: (r, 0, 0)),
        out_shape=jax.ShapeDtypeStruct((R_, T, NHASH), jnp.int32),
        scratch_shapes=[
            pltpu.VMEM((T, NHASH * BN), jnp.float32),
            pltpu.VMEM((T, NHASH * BN), jnp.float32),
        ],
    )(qkh, Rcat)


def _attn_body(qk_m, qk_p, v_m, v_p, tokc_ref, tokkv_ref, out_ref, lse_ref):
    W = (CPB + 1) * BUCKET
    qk_kv = jnp.concatenate([qk_p[0], qk_m[0]], axis=0)
    norm = jnp.sqrt(jnp.sum(qk_kv * qk_kv, axis=1, keepdims=True))
    k_kv = qk_kv / norm
    v_kv = jnp.concatenate([v_p[0], v_m[0]], axis=0)
    q = qk_m[0]
    s = lax.dot_general(q, k_kv, (((1,), (1,)), ((), ())),
                        preferred_element_type=jnp.float32) * (float(D) ** -0.5)
    tq = tokc_ref[0]
    tkv = tokkv_ref[0, 0]
    selfm = (tq == tkv).astype(jnp.float32)
    s = s * (1.0 - selfm) + selfm * (-1e5)
    ri = lax.broadcasted_iota(jnp.int32, (CPB * BUCKET, W), 0) // BUCKET
    ci = lax.broadcasted_iota(jnp.int32, (CPB * BUCKET, W), 1) // BUCKET
    band = (ci - ri >= 0) & (ci - ri <= 1)
    s = jnp.where(band, s, -1e30)
    m = jnp.max(s, axis=1, keepdims=True)
    e = jnp.exp(s - m)
    ssum = jnp.sum(e, axis=1, keepdims=True)
    lse = m + jnp.log(ssum)
    w = e / ssum
    out_ref[0] = lax.dot_general(w, v_kv, (((1,), (0,)), ((), ())),
                                 preferred_element_type=jnp.float32)
    lse_ref[0] = lse


def _attention(sorted_qk, sorted_v, tokC, tokKV):
    NB = CHUNKS // CPB
    return pl.pallas_call(
        _attn_body,
        grid=(R_, NB),
        in_specs=[
            pl.BlockSpec((1, CPB * BUCKET, D), lambda r, c: (r, c, 0)),
            pl.BlockSpec((1, BUCKET, D), lambda r, c: (r, (c * CPB - 1) % CHUNKS, 0)),
            pl.BlockSpec((1, CPB * BUCKET, D), lambda r, c: (r, c, 0)),
            pl.BlockSpec((1, BUCKET, D), lambda r, c: (r, (c * CPB - 1) % CHUNKS, 0)),
            pl.BlockSpec((1, CPB * BUCKET, 1), lambda r, c: (r, c, 0)),
            pl.BlockSpec((1, 1, 1, (CPB + 1) * BUCKET), lambda r, c: (r, c, 0, 0)),
        ],
        out_specs=[
            pl.BlockSpec((1, CPB * BUCKET, D), lambda r, c: (r, c, 0)),
            pl.BlockSpec((1, CPB * BUCKET, 1), lambda r, c: (r, c, 0)),
        ],
        out_shape=[
            jax.ShapeDtypeStruct((R_, NS, D), jnp.float32),
            jax.ShapeDtypeStruct((R_, NS, 1), jnp.float32),
        ],
    )(sorted_qk, sorted_qk, sorted_v, sorted_v, tokC, tokKV)


def _combine_body(qkv_ref, lg_ref, out_ref):
    lg = lg_ref[0]
    m = jnp.max(lg, axis=1, keepdims=True)
    lse4 = m + jnp.log(jnp.sum(jnp.exp(lg - m), axis=1, keepdims=True))
    qkv = qkv_ref[0]
    acc = jnp.zeros((qkv.shape[0], D), jnp.float32)
    for h in range(NHASH):
        ratio = jnp.exp(lg[:, h:h + 1] - lse4)
        acc = acc + qkv[:, h * D:(h + 1) * D] * ratio
    out_ref[0] = acc


def _combine(qkv_t, logits_t):
    TB = 512
    return pl.pallas_call(
        _combine_body,
        grid=(R_, T // TB),
        in_specs=[
            pl.BlockSpec((1, TB, NHASH * D), lambda r, t: (r, t, 0)),
            pl.BlockSpec((1, TB, NHASH), lambda r, t: (r, t, 0)),
        ],
        out_specs=pl.BlockSpec((1, TB, D), lambda r, t: (r, t, 0)),
        out_shape=jax.ShapeDtypeStruct((R_, T, D), jnp.float32),
    )(qkv_t, logits_t)


def _wo_body(a_ref, wo_ref, bo_ref, x1_ref, out_ref):
    out_ref[...] = (jnp.dot(a_ref[...], wo_ref[...], preferred_element_type=jnp.float32)
                    + bo_ref[...] + x1_ref[...])


def _wo_res(a_flat, Wo, bo, x1_flat):
    N = B * T
    TB = 512
    return pl.pallas_call(
        _wo_body,
        grid=(N // TB,),
        in_specs=[
            pl.BlockSpec((TB, EMB), lambda i: (i, 0)),
            pl.BlockSpec((EMB, EMB), lambda i: (0, 0)),
            pl.BlockSpec((1, EMB), lambda i: (0, 0)),
            pl.BlockSpec((TB, EMB), lambda i: (i, 0)),
        ],
        out_specs=pl.BlockSpec((TB, EMB), lambda i: (i, 0)),
        out_shape=jax.ShapeDtypeStruct((N, EMB), jnp.float32),
    )(a_flat, Wo, bo.reshape(1, EMB), x1_flat)


def _ffn_body(y1_ref, g_ref, be_ref, w1_ref, b1_ref, w2_ref, b2_ref, x2_ref, out_ref):
    x = y1_ref[...]
    mu = jnp.mean(x, axis=1, keepdims=True)
    var = jnp.mean((x - mu) ** 2, axis=1, keepdims=True)
    xn = g_ref[...] * (x - mu) / jnp.sqrt(var + 1e-3) + be_ref[...]
    h = jnp.maximum(jnp.dot(xn, w1_ref[...], preferred_element_type=jnp.float32) + b1_ref[...], 0.0)
    h2 = jnp.dot(h, w2_ref[...], preferred_element_type=jnp.float32) + b2_ref[...]
    h3 = jnp.maximum(jnp.dot(h2, w1_ref[...], preferred_element_type=jnp.float32) + b1_ref[...], 0.0)
    out_ref[...] = (jnp.dot(h3, w2_ref[...], preferred_element_type=jnp.float32)
                    + b2_ref[...] + x2_ref[...])


def _ffn2(y1_flat, g, be, W1, b1, W2, b2, x2_flat):
    N = B * T
    TB = 256
    H = 4 * EMB
    return pl.pallas_call(
        _ffn_body,
        grid=(N // TB,),
        in_specs=[
            pl.BlockSpec((TB, EMB), lambda i: (i, 0)),
            pl.BlockSpec((1, EMB), lambda i: (0, 0)),
            pl.BlockSpec((1, EMB), lambda i: (0, 0)),
            pl.BlockSpec((EMB, H), lambda i: (0, 0)),
            pl.BlockSpec((1, H), lambda i: (0, 0)),
            pl.BlockSpec((H, EMB), lambda i: (0, 0)),
            pl.BlockSpec((1, EMB), lambda i: (0, 0)),
            pl.BlockSpec((TB, EMB), lambda i: (i, 0)),
        ],
        out_specs=pl.BlockSpec((TB, EMB), lambda i: (i, 0)),
        out_shape=jax.ShapeDtypeStruct((N, EMB), jnp.float32),
    )(y1_flat, g.reshape(1, EMB), be.reshape(1, EMB), W1, b1.reshape(1, H),
      W2, b2.reshape(1, EMB), x2_flat)


def _mh_lsh(x2, Wk, Wv, Wo, bo, key, x1):
    qkh, vh = _proj(x2, Wk, Wv)
    Rmat = jnp.concatenate(
        [jax.random.normal(jax.random.fold_in(key, i), (B, D, NHASH, RSZ), dtype=jnp.float32)
         for i in range(HEADS)], axis=0)
    Rcat = jnp.concatenate([Rmat, -Rmat], axis=-1).reshape(R_, D, NHASH * BN)
    undo = _hashsort(qkh, Rcat)

    undo_flat = undo.reshape(R_, NS)
    sorted_tok = (jnp.argsort(undo_flat, axis=-1) // NHASH).astype(jnp.int32)

    sorted_qk = jnp.take_along_axis(qkh, sorted_tok[..., None], axis=1)
    sorted_v = jnp.take_along_axis(vh, sorted_tok[..., None], axis=1)

    tokf = sorted_tok.astype(jnp.float32)
    tokC = tokf.reshape(R_, NS, 1)
    rolled = jnp.concatenate([tokf[:, NS - BUCKET:], tokf], axis=1)
    NB = CHUNKS // CPB
    W = (CPB + 1) * BUCKET
    tokKV = jnp.stack([rolled[:, CPB * BUCKET * c: CPB * BUCKET * c + W]
                       for c in range(NB)], axis=1).reshape(R_, NB, 1, W)

    sorted_qkv, lse_o = _attention(sorted_qk, sorted_v, tokC, tokKV)
    lse_row = lse_o.reshape(R_, NS)

    qkv_t = jnp.take_along_axis(sorted_qkv, undo_flat[..., None], axis=1)
    qkv_t = qkv_t.reshape(R_, T, NHASH * D)
    logits_t = jnp.take_along_axis(lse_row, undo_flat, axis=1).reshape(R_, T, NHASH)

    attn_out = _combine(qkv_t, logits_t)
    out = jnp.transpose(attn_out.reshape(B, T, HEADS, D), (0, 2, 1, 3)).reshape(B, T, EMB)
    y1 = _wo_res(out.reshape(B * T, EMB), Wo, bo, x1.reshape(B * T, EMB))
    return y1.reshape(B, T, EMB)


def kernel(x, Wk0, Wv0, Wo0, bo0, g0, be0, W1_0, b1_0, W2_0, b2_0, Wk1, Wv1, Wo1, bo1, g1, be1, W1_1, b1_1, W2_1, b2_1):
    params = [
        (Wk0, Wv0, Wo0, bo0, g0, be0, W1_0, b1_0, W2_0, b2_0),
        (Wk1, Wv1, Wo1, bo1, g1, be1, W1_1, b1_1, W2_1, b2_1),
    ]
    key = jax.random.key(42)
    x1, x2 = x, x
    for d, (Wk, Wv, Wo, bo, g, be, W1, b1, W2, b2) in enumerate(params):
        y1 = _mh_lsh(x2, Wk, Wv, Wo, bo, jax.random.fold_in(key, d), x1)
        y2 = _ffn2(y1.reshape(B * T, EMB), g, be, W1, b1, W2, b2,
                   x2.reshape(B * T, EMB)).reshape(B, T, EMB)
        x1, x2 = y1, y2
    return jnp.concatenate([x1, x2], axis=-1)


# speedup vs baseline: 5.1527x; 5.1527x over previous
import functools

import jax
import jax.numpy as jnp
from jax import lax
from jax.experimental import pallas as pl
from jax.experimental.pallas import tpu as pltpu
from jax.experimental.pallas import tpu_sc as plsc

B, T, EMB = 2, 2048, 768
HEADS, DEPTH = 12, 2
BUCKET, NHASH, FF_CHUNKS = 64, 4, 16
D = EMB // HEADS
BN = T // BUCKET
CHUNKS = BN * NHASH
R_ = B * HEADS
RSZ = BN // 2
NS = NHASH * T
CPB = 8


def _proj_body(x_ref, wk_ref, wv_ref, qkv_ref):
    x = x_ref[0]
    qk = jnp.dot(x, wk_ref[0], preferred_element_type=jnp.float32)
    v = jnp.dot(x, wv_ref[0], preferred_element_type=jnp.float32)
    qkv_ref[0] = jnp.concatenate([qk, v], axis=1)


def _proj(x2, Wk, Wv):
    TB = 512
    Wkh = Wk.reshape(EMB, HEADS, D).transpose(1, 0, 2)
    Wvh = Wv.reshape(EMB, HEADS, D).transpose(1, 0, 2)
    return pl.pallas_call(
        _proj_body,
        grid=(R_, T // TB),
        in_specs=[
            pl.BlockSpec((1, TB, EMB), lambda r, t: (r // HEADS, t, 0)),
            pl.BlockSpec((1, EMB, D), lambda r, t: (r % HEADS, 0, 0)),
            pl.BlockSpec((1, EMB, D), lambda r, t: (r % HEADS, 0, 0)),
        ],
        out_specs=pl.BlockSpec((1, TB, 2 * D), lambda r, t: (r, t, 0)),
        out_shape=jax.ShapeDtypeStruct((R_, T, 2 * D), jnp.float32),
    )(x2, Wkh, Wvh)


def _hashsort_body(qk_ref, rcat_ref, undo_ref, oh_ref, cum_ref):
    qk = qk_ref[0][:, :D]
    xr = jnp.dot(qk, rcat_ref[0], preferred_element_type=jnp.float32)
    il = lax.broadcasted_iota(jnp.int32, (T, BN), 1)
    for h in range(NHASH):
        sub = xr[:, h * BN:(h + 1) * BN]
        m = jnp.max(sub, axis=1, keepdims=True)
        idx = jnp.min(jnp.where(sub == m, il, BN + 1), axis=1, keepdims=True)
        oh_ref[:, h * BN:(h + 1) * BN] = (il == idx).astype(jnp.float32)
    TB = 256
    r_i = lax.broadcasted_iota(jnp.int32, (TB, TB), 0)
    c_i = lax.broadcasted_iota(jnp.int32, (TB, TB), 1)
    L = (c_i <= r_i).astype(jnp.float32)
    carry = jnp.zeros((1, NHASH * BN), jnp.float32)
    for b in range(T // TB):
        blk = oh_ref[b * TB:(b + 1) * TB, :]
        inc = jnp.dot(L, blk, preferred_element_type=jnp.float32)
        cum_ref[b * TB:(b + 1) * TB, :] = inc + carry
        carry = carry + inc[TB - 1:TB, :]
    g_r = lax.broadcasted_iota(jnp.int32, (NHASH * BN, NHASH * BN), 0)
    g_c = lax.broadcasted_iota(jnp.int32, (NHASH * BN, NHASH * BN), 1)
    M = ((g_r // BN == g_c // BN) & (g_r < g_c)).astype(jnp.float32)
    offs = jnp.dot(carry, M, preferred_element_type=jnp.float32)
    cols = []
    for h in range(NHASH):
        oh_h = oh_ref[:, h * BN:(h + 1) * BN]
        cum_h = cum_ref[:, h * BN:(h + 1) * BN]
        rank_incl = jnp.sum(cum_h * oh_h, axis=1, keepdims=True)
        offpick = jnp.sum(offs[:, h * BN:(h + 1) * BN] * oh_h, axis=1, keepdims=True)
        dest = rank_incl - 1.0 + offpick + float(T) * h
        cols.append(dest.astype(jnp.int32))
    undo_ref[0] = jnp.concatenate(cols, axis=1)


def _hashsort(qkh, Rcat):
    return pl.pallas_call(
        _hashsort_body,
        grid=(R_,),
        in_specs=[
            pl.BlockSpec((1, T, 2 * D), lambda r: (r, 0, 0)),
            pl.BlockSpec((1, D, NHASH * BN), lambda r: (r, 0, 0)),
        ],
        out_specs=pl.BlockSpec((1, T, NHASH), lambda r: (r, 0, 0)),
        out_shape=jax.ShapeDtypeStruct((R_, T, NHASH), jnp.int32),
        scratch_shapes=[
            pltpu.VMEM((T, NHASH * BN), jnp.float32),
            pltpu.VMEM((T, NHASH * BN), jnp.float32),
        ],
    )(qkh, Rcat)


def _attn_body(qkv_m, qkv_p, tokc_ref, tokkv_ref, out_ref):
    W = (CPB + 1) * BUCKET
    kv_all = jnp.concatenate([qkv_p[0], qkv_m[0]], axis=0)
    qk_kv = kv_all[:, :D]
    v_kv = kv_all[:, D:]
    norm = jnp.sqrt(jnp.sum(qk_kv * qk_kv, axis=1, keepdims=True))
    k_kv = qk_kv / norm
    q = qkv_m[0][:, :D]
    s = lax.dot_general(q, k_kv, (((1,), (1,)), ((), ())),
                        preferred_element_type=jnp.float32) * (float(D) ** -0.5)
    tq = tokc_ref[0]
    tkv = tokkv_ref[0, 0]
    selfm = (tq == tkv).astype(jnp.float32)
    s = s * (1.0 - selfm) + selfm * (-1e5)
    ri = lax.broadcasted_iota(jnp.int32, (CPB * BUCKET, W), 0) // BUCKET
    ci = lax.broadcasted_iota(jnp.int32, (CPB * BUCKET, W), 1) // BUCKET
    band = (ci - ri >= 0) & (ci - ri <= 1)
    s = jnp.where(band, s, -1e30)
    m = jnp.max(s, axis=1, keepdims=True)
    e = jnp.exp(s - m)
    ssum = jnp.sum(e, axis=1, keepdims=True)
    lse = m + jnp.log(ssum)
    w = e / ssum
    o = lax.dot_general(w, v_kv, (((1,), (0,)), ((), ())),
                        preferred_element_type=jnp.float32)
    out_ref[0] = jnp.concatenate(
        [o, lse, jnp.zeros((CPB * BUCKET, D - 1), jnp.float32)], axis=1)


def _attention(sorted_qkv_in, tokC, tokKV):
    NB = CHUNKS // CPB
    return pl.pallas_call(
        _attn_body,
        grid=(R_, NB),
        in_specs=[
            pl.BlockSpec((1, CPB * BUCKET, 2 * D), lambda r, c: (r, c, 0)),
            pl.BlockSpec((1, BUCKET, 2 * D), lambda r, c: (r, (c * CPB - 1) % CHUNKS, 0)),
            pl.BlockSpec((1, CPB * BUCKET, 1), lambda r, c: (r, c, 0)),
            pl.BlockSpec((1, 1, 1, (CPB + 1) * BUCKET), lambda r, c: (r, c, 0, 0)),
        ],
        out_specs=pl.BlockSpec((1, CPB * BUCKET, 2 * D), lambda r, c: (r, c, 0)),
        out_shape=jax.ShapeDtypeStruct((R_, NS, 2 * D), jnp.float32),
    )(sorted_qkv_in, sorted_qkv_in, tokC, tokKV)


def _combine_body(qkvl_ref, out_ref):
    qkvl = qkvl_ref[0]
    lg = jnp.concatenate(
        [qkvl[:, h * 2 * D + D: h * 2 * D + D + 1] for h in range(NHASH)], axis=1)
    m = jnp.max(lg, axis=1, keepdims=True)
    lse4 = m + jnp.log(jnp.sum(jnp.exp(lg - m), axis=1, keepdims=True))
    acc = jnp.zeros((qkvl.shape[0], D), jnp.float32)
    for h in range(NHASH):
        ratio = jnp.exp(lg[:, h:h + 1] - lse4)
        acc = acc + qkvl[:, h * 2 * D: h * 2 * D + D] * ratio
    out_ref[0] = acc


def _combine(qkvl_t):
    TB = 512
    return pl.pallas_call(
        _combine_body,
        grid=(R_, T // TB),
        in_specs=[
            pl.BlockSpec((1, TB, NHASH * 2 * D), lambda r, t: (r, t, 0)),
        ],
        out_specs=pl.BlockSpec((1, TB, D), lambda r, t: (r, t, 0)),
        out_shape=jax.ShapeDtypeStruct((R_, T, D), jnp.float32),
    )(qkvl_t)


def _wo_body(a_ref, wo_ref, bo_ref, x1_ref, out_ref):
    out_ref[...] = (jnp.dot(a_ref[...], wo_ref[...], preferred_element_type=jnp.float32)
                    + bo_ref[...] + x1_ref[...])


def _wo_res(a_flat, Wo, bo, x1_flat):
    N = B * T
    TB = 512
    return pl.pallas_call(
        _wo_body,
        grid=(N // TB,),
        in_specs=[
            pl.BlockSpec((TB, EMB), lambda i: (i, 0)),
            pl.BlockSpec((EMB, EMB), lambda i: (0, 0)),
            pl.BlockSpec((1, EMB), lambda i: (0, 0)),
            pl.BlockSpec((TB, EMB), lambda i: (i, 0)),
        ],
        out_specs=pl.BlockSpec((TB, EMB), lambda i: (i, 0)),
        out_shape=jax.ShapeDtypeStruct((N, EMB), jnp.float32),
    )(a_flat, Wo, bo.reshape(1, EMB), x1_flat)


def _ffn_body(y1_ref, g_ref, be_ref, w1_ref, b1_ref, w2_ref, b2_ref, x2_ref, out_ref):
    x = y1_ref[...]
    mu = jnp.mean(x, axis=1, keepdims=True)
    var = jnp.mean((x - mu) ** 2, axis=1, keepdims=True)
    xn = g_ref[...] * (x - mu) / jnp.sqrt(var + 1e-3) + be_ref[...]
    h = jnp.maximum(jnp.dot(xn, w1_ref[...], preferred_element_type=jnp.float32) + b1_ref[...], 0.0)
    h2 = jnp.dot(h, w2_ref[...], preferred_element_type=jnp.float32) + b2_ref[...]
    h3 = jnp.maximum(jnp.dot(h2, w1_ref[...], preferred_element_type=jnp.float32) + b1_ref[...], 0.0)
    out_ref[...] = (jnp.dot(h3, w2_ref[...], preferred_element_type=jnp.float32)
                    + b2_ref[...] + x2_ref[...])


def _ffn2(y1_flat, g, be, W1, b1, W2, b2, x2_flat):
    N = B * T
    TB = 256
    H = 4 * EMB
    return pl.pallas_call(
        _ffn_body,
        grid=(N // TB,),
        in_specs=[
            pl.BlockSpec((TB, EMB), lambda i: (i, 0)),
            pl.BlockSpec((1, EMB), lambda i: (0, 0)),
            pl.BlockSpec((1, EMB), lambda i: (0, 0)),
            pl.BlockSpec((EMB, H), lambda i: (0, 0)),
            pl.BlockSpec((1, H), lambda i: (0, 0)),
            pl.BlockSpec((H, EMB), lambda i: (0, 0)),
            pl.BlockSpec((1, EMB), lambda i: (0, 0)),
            pl.BlockSpec((TB, EMB), lambda i: (i, 0)),
        ],
        out_specs=pl.BlockSpec((TB, EMB), lambda i: (i, 0)),
        out_shape=jax.ShapeDtypeStruct((N, EMB), jnp.float32),
    )(y1_flat, g.reshape(1, EMB), be.reshape(1, EMB), W1, b1.reshape(1, H),
      W2, b2.reshape(1, EMB), x2_flat)


NWORK = 32


def _sc_gather(table, idx):
    NI = idx.shape[0]
    SEG = 512
    PER = NI // (SEG * NWORK)

    @functools.partial(
        pl.kernel,
        mesh=plsc.VectorSubcoreMesh(core_axis_name="c", subcore_axis_name="s"),
        out_type=jax.ShapeDtypeStruct((NI, 2 * D), jnp.float32),
        scratch_types=[pltpu.VMEM((SEG,), jnp.int32),
                       pltpu.VMEM((SEG, 2 * D), jnp.float32),
                       pltpu.SemaphoreType.DMA],
    )
    def gk(ta, idx_ref, oa, idx_v, ra, sa):
        wid = lax.axis_index("s") * 2 + lax.axis_index("c")

        def body(i, carry):
            base = (wid * PER + i) * SEG
            pltpu.sync_copy(idx_ref.at[pl.ds(base, SEG)], idx_v)
            pltpu.async_copy(ta.at[idx_v], ra, sa).wait()
            pltpu.sync_copy(ra, oa.at[pl.ds(base, SEG)])
            return carry

        lax.fori_loop(0, PER, body, 0)

    return gk(table, idx)


def _mh_lsh(x2, Wk, Wv, Wo, bo, key, x1):
    qkvh = _proj(x2, Wk, Wv)
    Rmat = jnp.concatenate(
        [jax.random.normal(jax.random.fold_in(key, i), (B, D, NHASH, RSZ), dtype=jnp.float32)
         for i in range(HEADS)], axis=0)
    Rcat = jnp.concatenate([Rmat, -Rmat], axis=-1).reshape(R_, D, NHASH * BN)
    undo = _hashsort(qkvh, Rcat)

    undo_flat = undo.reshape(R_, NS)
    sorted_tok = (jnp.argsort(undo_flat, axis=-1) // NHASH).astype(jnp.int32)

    stok_g = (sorted_tok + (jnp.arange(R_, dtype=jnp.int32) * T)[:, None]).reshape(R_ * NS)
    sorted_qkv_in = _sc_gather(qkvh.reshape(R_ * T, 2 * D), stok_g).reshape(R_, NS, 2 * D)

    tokf = sorted_tok.astype(jnp.float32)
    tokC = tokf.reshape(R_, NS, 1)
    rolled = jnp.concatenate([tokf[:, NS - BUCKET:], tokf], axis=1)
    NB = CHUNKS // CPB
    W = (CPB + 1) * BUCKET
    tokKV = jnp.stack([rolled[:, CPB * BUCKET * c: CPB * BUCKET * c + W]
                       for c in range(NB)], axis=1).reshape(R_, NB, 1, W)

    sorted_qkvl = _attention(sorted_qkv_in, tokC, tokKV)

    undo_g = (undo_flat + (jnp.arange(R_, dtype=jnp.int32) * NS)[:, None]).reshape(R_ * NS)
    qkvl_t = _sc_gather(sorted_qkvl.reshape(R_ * NS, 2 * D), undo_g).reshape(R_, T, NHASH * 2 * D)

    attn_out = _combine(qkvl_t)
    out = jnp.transpose(attn_out.reshape(B, T, HEADS, D), (0, 2, 1, 3)).reshape(B, T, EMB)
    y1 = _wo_res(out.reshape(B * T, EMB), Wo, bo, x1.reshape(B * T, EMB))
    return y1.reshape(B, T, EMB)


def kernel(x, Wk0, Wv0, Wo0, bo0, g0, be0, W1_0, b1_0, W2_0, b2_0, Wk1, Wv1, Wo1, bo1, g1, be1, W1_1, b1_1, W2_1, b2_1):
    params = [
        (Wk0, Wv0, Wo0, bo0, g0, be0, W1_0, b1_0, W2_0, b2_0),
        (Wk1, Wv1, Wo1, bo1, g1, be1, W1_1, b1_1, W2_1, b2_1),
    ]
    key = jax.random.key(42)
    x1, x2 = x, x
    for d, (Wk, Wv, Wo, bo, g, be, W1, b1, W2, b2) in enumerate(params):
        y1 = _mh_lsh(x2, Wk, Wv, Wo, bo, jax.random.fold_in(key, d), x1)
        y2 = _ffn2(y1.reshape(B * T, EMB), g, be, W1, b1, W2, b2,
                   x2.reshape(B * T, EMB)).reshape(B, T, EMB)
        x1, x2 = y1, y2
    return jnp.concatenate([x1, x2], axis=-1)
